# baseline (speedup 1.0000x reference)
import functools

import jax
import jax.numpy as jnp
from jax import lax
from jax.experimental import pallas as pl
from jax.experimental.pallas import tpu as pltpu
from jax.experimental.pallas import tpu_sc as plsc

G = 8
K = 8192
E = 64
BN = 4
T = 1024
NTOK = BN * T
CHUNK = 2048
NCH = K // CHUNK
NLOSS = float(NTOK * G * E)

SCL = 2.0 ** 50
ISCL = 2.0 ** -13
IINV = 2.0 ** 13

SC_CORES = 2
SC_SUBCORES = 16
NW = SC_CORES * SC_SUBCORES
IDX_CH = 128
CH_PER_W = (G * NTOK) // (NW * IDX_CH)

_CONTRACT_ET = (((0,), (1,)), ((), ()))


def _argmin_codes(xb, w_ref):
    ones_row = jnp.ones((1, E), jnp.float32)
    xsq = (xb * SCL) * xb
    a = lax.dot_general(xsq, ones_row, _CONTRACT_ET,
                        preferred_element_type=jnp.float32)
    best = jnp.full((T, 1), jnp.inf, jnp.float32)
    kbest = jnp.zeros((T, 1), jnp.float32)
    for c in range(NCH):
        wc = w_ref[0, c * CHUNK:(c + 1) * CHUNK, :]
        bsq = lax.dot_general(ones_row, (wc * SCL) * wc,
                              (((1,), (1,)), ((), ())),
                              preferred_element_type=jnp.float32)
        s2 = lax.dot_general(xb, (-2.0 * SCL) * wc, _CONTRACT_ET,
                             preferred_element_type=jnp.float32)
        dd = (a + bsq) + s2
        m = jnp.min(dd, axis=1, keepdims=True)
        iota_s = (lax.broadcasted_iota(jnp.int32, (1, CHUNK), 1)
                  .astype(jnp.float32) + float(c * CHUNK)) * ISCL
        cand = (dd - m) + iota_s
        kc = jnp.min(cand, axis=1, keepdims=True)
        upd = m < best
        kbest = jnp.where(upd, kc, kbest)
        best = jnp.where(upd, m, best)
    return (kbest * IINV).astype(jnp.int32)


def _tc1_body(x_ref, w_ref, idx_ref, gidx_ref):
    g = pl.program_id(0)
    xb = x_ref[0, 0, :, :]
    bidx = _argmin_codes(xb, w_ref)
    idx_ref[0, 0, :, :] = bidx
    gidx_ref[0, 0, :, :] = bidx + g * K


def _tc2_body(x_ref, zq_ref, w_ref, idx_ref, gidx_ref, q1_ref, s_ref):
    g = pl.program_id(0)
    b = pl.program_id(1)
    xb = x_ref[0, 0, :, :]
    zq = jnp.transpose(zq_ref[0, 0, :, :])
    diff = zq - xb
    q1 = xb + diff
    r = xb - q1

    @pl.when((g == 0) & (b == 0))
    def _init():
        s_ref[0, 0] = 0.0

    s_ref[0, 0] += jnp.sum(diff * diff)
    q1_ref[0, 0, :, :] = q1
    bidx = _argmin_codes(r, w_ref)
    idx_ref[0, 0, :, :] = bidx
    gidx_ref[0, 0, :, :] = bidx + g * K


def _tc3_body(x_ref, q1_ref, zq2_ref, qs_ref, s_ref):
    g = pl.program_id(0)
    b = pl.program_id(1)
    xb = x_ref[0, 0, :, :]
    q1 = q1_ref[0, 0, :, :]
    zq2 = jnp.transpose(zq2_ref[0, 0, :, :])
    r = xb - q1
    diff2 = zq2 - r
    q2 = r + diff2

    @pl.when((g == 0) & (b == 0))
    def _init():
        s_ref[0, 0] = 0.0

    s_ref[0, 0] += jnp.sum(diff2 * diff2)
    qs_ref[0, 0, :, :] = q1 + q2


_X_SPEC = pl.BlockSpec((1, 1, E, T), lambda g, b: (b, g, 0, 0))
_W_SPEC = pl.BlockSpec((1, K, E), lambda g, b: (g, 0, 0))
_ZQ_SPEC = pl.BlockSpec((1, 1, T, E), lambda g, b: (g, b, 0, 0))
_IDX_SPEC = pl.BlockSpec((1, 1, T, 1), lambda g, b: (g, b, 0, 0))
_Q_SPEC = pl.BlockSpec((1, 1, E, T), lambda g, b: (b, g, 0, 0))
_S_SPEC = pl.BlockSpec(memory_space=pltpu.SMEM, block_shape=(1, 1),
                       index_map=lambda g, b: (0, 0))

_IDX_TY = jax.ShapeDtypeStruct((G, BN, T, 1), jnp.int32)
_Q_TY = jax.ShapeDtypeStruct((BN, G, E, T), jnp.float32)
_S_TY = jax.ShapeDtypeStruct((1, 1), jnp.float32)


def _tc1(xg, w):
    return pl.pallas_call(
        _tc1_body,
        grid=(G, BN),
        in_specs=[_X_SPEC, _W_SPEC],
        out_specs=[_IDX_SPEC, _IDX_SPEC],
        out_shape=[_IDX_TY, _IDX_TY],
    )(xg, w)


def _tc2(xg, zq1, w2):
    return pl.pallas_call(
        _tc2_body,
        grid=(G, BN),
        in_specs=[_X_SPEC, _ZQ_SPEC, _W_SPEC],
        out_specs=[_IDX_SPEC, _IDX_SPEC, _Q_SPEC, _S_SPEC],
        out_shape=[_IDX_TY, _IDX_TY, _Q_TY, _S_TY],
    )(xg, zq1, w2)


def _tc3(xg, q1, zq2):
    return pl.pallas_call(
        _tc3_body,
        grid=(G, BN),
        in_specs=[_X_SPEC, _Q_SPEC, _ZQ_SPEC],
        out_specs=[_Q_SPEC, _S_SPEC],
        out_shape=[_Q_TY, _S_TY],
    )(xg, q1, zq2)


def _sc_gather_body(table_hbm, idx_hbm, out_hbm, idx_v, rows_v, sem):
    wid = lax.axis_index("s") * SC_CORES + lax.axis_index("c")
    base = wid * CH_PER_W
    pltpu.sync_copy(idx_hbm.at[pl.ds(base, CH_PER_W)], idx_v)
    copies = [
        pltpu.async_copy(table_hbm.at[idx_v.at[j]], rows_v.at[j], sem)
        for j in range(CH_PER_W)
    ]
    for cp in copies:
        cp.wait()
    pltpu.sync_copy(rows_v, out_hbm.at[pl.ds(base, CH_PER_W)])


def _sc_gather(table, gidx2d):
    call = pl.kernel(
        _sc_gather_body,
        out_type=jax.ShapeDtypeStruct((NW * CH_PER_W, IDX_CH, E), jnp.float32),
        mesh=plsc.VectorSubcoreMesh(core_axis_name="c", subcore_axis_name="s"),
        scratch_types=[
            pltpu.VMEM((CH_PER_W, IDX_CH), jnp.int32),
            pltpu.VMEM((CH_PER_W, IDX_CH, E), jnp.float32),
            pltpu.SemaphoreType.DMA,
        ],
        compiler_params=pltpu.CompilerParams(use_tc_tiling_on_sc=False),
    )
    return call(table, gidx2d)


def kernel(x, codebooks1, codebooks2):
    xg = x.reshape(BN, G, E, T)

    idx1, gidx1 = _tc1(xg, codebooks1)
    zq1 = _sc_gather(codebooks1.reshape(G * K, E),
                     gidx1.reshape(NW * CH_PER_W, IDX_CH))
    idx2, gidx2, q1, s1 = _tc2(xg, zq1.reshape(G, BN, T, E), codebooks2)
    zq2 = _sc_gather(codebooks2.reshape(G * K, E),
                     gidx2.reshape(NW * CH_PER_W, IDX_CH))
    qs, s2 = _tc3(xg, q1, zq2.reshape(G, BN, T, E))

    quantized_out = qs.reshape(BN, G * E, T)

    m1 = s1[0, 0] / NLOSS
    m2 = s2[0, 0] / NLOSS
    l1 = 1.0 * m1 + 0.25 * m1
    l2 = 1.0 * m2 + 0.25 * m2
    total_loss = (l1 + l2) / 2.0

    i1 = idx1.reshape(G, NTOK)
    i2 = idx2.reshape(G, NTOK)
    out = (quantized_out, total_loss)
    out += tuple(i1[g] for g in range(G))
    out += tuple(i2[g] for g in range(G))
    return out


# speedup vs baseline: 1.0295x; 1.0295x over previous
import functools

import jax
import jax.numpy as jnp
from jax import lax
from jax.experimental import pallas as pl
from jax.experimental.pallas import tpu as pltpu
from jax.experimental.pallas import tpu_sc as plsc

G = 8
K = 8192
E = 64
BN = 4
HB = 2
BH = BN // HB
T = 1024
NTOK = BN * T
CHUNK = 1024
NCH = K // CHUNK
NLOSS = float(NTOK * G * E)

SCL = 2.0 ** 50
ISCL = 2.0 ** -13
IINV = 2.0 ** 13

SC_CORES = 2
SC_SUBCORES = 16
NW = SC_CORES * SC_SUBCORES
IDX_CH = 128

_CONTRACT_ET = (((0,), (1,)), ((), ()))


def _argmin_codes(xb, w_ref):
    ones_row = jnp.ones((1, E), jnp.float32)
    xsq = (xb * SCL) * xb
    a = lax.dot_general(xsq, ones_row, _CONTRACT_ET,
                        preferred_element_type=jnp.float32)
    best = jnp.full((T, 1), jnp.inf, jnp.float32)
    kbest = jnp.zeros((T, 1), jnp.float32)
    for c in range(NCH):
        wc = w_ref[0, c * CHUNK:(c + 1) * CHUNK, :]
        bsq = lax.dot_general(ones_row, (wc * SCL) * wc,
                              (((1,), (1,)), ((), ())),
                              preferred_element_type=jnp.float32)
        s2 = lax.dot_general(xb, (-2.0 * SCL) * wc, _CONTRACT_ET,
                             preferred_element_type=jnp.float32)
        dd = (a + bsq) + s2
        m = jnp.min(dd, axis=1, keepdims=True)
        iota_s = (lax.broadcasted_iota(jnp.int32, (1, CHUNK), 1)
                  .astype(jnp.float32) + float(c * CHUNK)) * ISCL
        cand = (dd - m) + iota_s
        kc = jnp.min(cand, axis=1, keepdims=True)
        upd = m < best
        kbest = jnp.where(upd, kc, kbest)
        best = jnp.where(upd, m, best)
    return (kbest * IINV).astype(jnp.int32)


def _tc1_body(x_ref, w_ref, idx_ref, gidx_ref):
    g = pl.program_id(0)
    xb = x_ref[0, 0, :, :]
    bidx = _argmin_codes(xb, w_ref)
    idx_ref[0, 0, :, :] = bidx
    gidx_ref[0, 0, :, :] = bidx + g * K


def _tc2_body(x_ref, zq_ref, w_ref, idx_ref, gidx_ref, q1_ref, s_ref):
    g = pl.program_id(0)
    b = pl.program_id(1)
    xb = x_ref[0, 0, :, :]
    zq = jnp.transpose(zq_ref[0, 0, :, :])
    diff = zq - xb
    q1 = xb + diff
    r = xb - q1

    @pl.when((g == 0) & (b == 0))
    def _init():
        s_ref[0, 0] = 0.0

    s_ref[0, 0] += jnp.sum(diff * diff)
    q1_ref[0, 0, :, :] = q1
    bidx = _argmin_codes(r, w_ref)
    idx_ref[0, 0, :, :] = bidx
    gidx_ref[0, 0, :, :] = bidx + g * K


def _tc3_body(x_ref, q1_ref, zq2_ref, qs_ref, s_ref):
    g = pl.program_id(0)
    b = pl.program_id(1)
    xb = x_ref[0, 0, :, :]
    q1 = q1_ref[0, 0, :, :]
    zq2 = jnp.transpose(zq2_ref[0, 0, :, :])
    r = xb - q1
    diff2 = zq2 - r
    q2 = r + diff2

    @pl.when((g == 0) & (b == 0))
    def _init():
        s_ref[0, 0] = 0.0

    s_ref[0, 0] += jnp.sum(diff2 * diff2)
    qs_ref[0, 0, :, :] = q1 + q2


def _x_spec(b0):
    return pl.BlockSpec((1, 1, E, T), lambda g, b: (b + b0, g, 0, 0))


_W_SPEC = pl.BlockSpec((1, K, E), lambda g, b: (g, 0, 0))
_ZQ_SPEC = pl.BlockSpec((1, 1, T, E), lambda g, b: (g, b, 0, 0))
_IDX_SPEC = pl.BlockSpec((1, 1, T, 1), lambda g, b: (g, b, 0, 0))
_QH_SPEC = pl.BlockSpec((1, 1, E, T), lambda g, b: (b, g, 0, 0))
_S_SPEC = pl.BlockSpec(memory_space=pltpu.SMEM, block_shape=(1, 1),
                       index_map=lambda g, b: (0, 0))

_IDX_TY = jax.ShapeDtypeStruct((G, BH, T, 1), jnp.int32)
_Q_TY = jax.ShapeDtypeStruct((BH, G, E, T), jnp.float32)
_S_TY = jax.ShapeDtypeStruct((1, 1), jnp.float32)


def _tc1(xg, w, b0):
    return pl.pallas_call(
        _tc1_body,
        grid=(G, BH),
        in_specs=[_x_spec(b0), _W_SPEC],
        out_specs=[_IDX_SPEC, _IDX_SPEC],
        out_shape=[_IDX_TY, _IDX_TY],
    )(xg, w)


def _tc2(xg, zq1, w2, b0):
    return pl.pallas_call(
        _tc2_body,
        grid=(G, BH),
        in_specs=[_x_spec(b0), _ZQ_SPEC, _W_SPEC],
        out_specs=[_IDX_SPEC, _IDX_SPEC, _QH_SPEC, _S_SPEC],
        out_shape=[_IDX_TY, _IDX_TY, _Q_TY, _S_TY],
    )(xg, zq1, w2)


def _tc3(xg, q1, zq2, b0):
    return pl.pallas_call(
        _tc3_body,
        grid=(G, BH),
        in_specs=[_x_spec(b0), _QH_SPEC, _ZQ_SPEC],
        out_specs=[_QH_SPEC, _S_SPEC],
        out_shape=[_Q_TY, _S_TY],
    )(xg, q1, zq2)


def _sc_gather_body(nch, table_hbm, idx_hbm, out_hbm, idx_v, rows_v, sem):
    wid = lax.axis_index("s") * SC_CORES + lax.axis_index("c")
    base = wid * nch
    pltpu.sync_copy(idx_hbm.at[pl.ds(base, nch)], idx_v)
    copies = [
        pltpu.async_copy(table_hbm.at[idx_v.at[j]], rows_v.at[j], sem)
        for j in range(nch)
    ]
    for cp in copies:
        cp.wait()
    pltpu.sync_copy(rows_v, out_hbm.at[pl.ds(base, nch)])


def _sc_gather(table, gidx2d):
    n = gidx2d.shape[0]
    nch = n // NW
    call = pl.kernel(
        functools.partial(_sc_gather_body, nch),
        out_type=jax.ShapeDtypeStruct((n, IDX_CH, E), jnp.float32),
        mesh=plsc.VectorSubcoreMesh(core_axis_name="c", subcore_axis_name="s"),
        scratch_types=[
            pltpu.VMEM((nch, IDX_CH), jnp.int32),
            pltpu.VMEM((nch, IDX_CH, E), jnp.float32),
            pltpu.SemaphoreType.DMA,
        ],
        compiler_params=pltpu.CompilerParams(use_tc_tiling_on_sc=False),
    )
    return call(table, gidx2d)


def kernel(x, codebooks1, codebooks2):
    xg = x.reshape(BN, G, E, T)
    tab1 = codebooks1.reshape(G * K, E)
    tab2 = codebooks2.reshape(G * K, E)
    nh = (G * BH * T) // IDX_CH

    idx1 = [None] * HB
    gidx1 = [None] * HB
    for h in range(HB):
        idx1[h], gidx1[h] = _tc1(xg, codebooks1, h * BH)
    zq1 = [_sc_gather(tab1, gidx1[h].reshape(nh, IDX_CH)) for h in range(HB)]
    idx2 = [None] * HB
    gidx2 = [None] * HB
    q1 = [None] * HB
    s1 = [None] * HB
    for h in range(HB):
        idx2[h], gidx2[h], q1[h], s1[h] = _tc2(
            xg, zq1[h].reshape(G, BH, T, E), codebooks2, h * BH)
    zq2 = [_sc_gather(tab2, gidx2[h].reshape(nh, IDX_CH)) for h in range(HB)]
    qs = [None] * HB
    s2 = [None] * HB
    for h in range(HB):
        qs[h], s2[h] = _tc3(xg, q1[h], zq2[h].reshape(G, BH, T, E), h * BH)

    quantized_out = jnp.concatenate(qs, axis=0).reshape(BN, G * E, T)

    m1 = (s1[0][0, 0] + s1[1][0, 0]) / NLOSS
    m2 = (s2[0][0, 0] + s2[1][0, 0]) / NLOSS
    l1 = 1.0 * m1 + 0.25 * m1
    l2 = 1.0 * m2 + 0.25 * m2
    total_loss = (l1 + l2) / 2.0

    i1 = jnp.concatenate(idx1, axis=1).reshape(G, NTOK)
    i2 = jnp.concatenate(idx2, axis=1).reshape(G, NTOK)
    out = (quantized_out, total_loss)
    out += tuple(i1[g] for g in range(G))
    out += tuple(i2[g] for g in range(G))
    return out


# speedup vs baseline: 1.0717x; 1.0410x over previous
import functools

import jax
import jax.numpy as jnp
from jax import lax
from jax.experimental import pallas as pl
from jax.experimental.pallas import tpu as pltpu
from jax.experimental.pallas import tpu_sc as plsc

G = 8
K = 8192
E = 64
BN = 4
T = 1024
NTOK = BN * T
CHUNK = 1024
NCH = K // CHUNK
NLOSS = float(NTOK * G * E)

SCL = 2.0 ** 50
ISCL = 2.0 ** -13
IINV = 2.0 ** 13

SC_CORES = 2
SC_SUBCORES = 16
NW = SC_CORES * SC_SUBCORES
IDX_CH = 128
CH_PER_W = (G * NTOK) // (NW * IDX_CH)

_CONTRACT_ET = (((0,), (1,)), ((), ()))


def _argmin_codes(xb, w_ref):
    ones_row = jnp.ones((1, E), jnp.float32)
    xsq = (xb * SCL) * xb
    a = lax.dot_general(xsq, ones_row, _CONTRACT_ET,
                        preferred_element_type=jnp.float32)
    best = jnp.full((T, 1), jnp.inf, jnp.float32)
    kbest = jnp.zeros((T, 1), jnp.float32)
    for c in range(NCH):
        wc = w_ref[0, c * CHUNK:(c + 1) * CHUNK, :]
        bsq = lax.dot_general(ones_row, (wc * SCL) * wc,
                              (((1,), (1,)), ((), ())),
                              preferred_element_type=jnp.float32)
        s2 = lax.dot_general(xb, (-2.0 * SCL) * wc, _CONTRACT_ET,
                             preferred_element_type=jnp.float32)
        dd = (a + bsq) + s2
        m = jnp.min(dd, axis=1, keepdims=True)
        iota_s = (lax.broadcasted_iota(jnp.int32, (1, CHUNK), 1)
                  .astype(jnp.float32) + float(c * CHUNK)) * ISCL
        cand = (dd - m) + iota_s
        kc = jnp.min(cand, axis=1, keepdims=True)
        upd = m < best
        kbest = jnp.where(upd, kc, kbest)
        best = jnp.where(upd, m, best)
    return (kbest * IINV).astype(jnp.int32)


def _tc1_body(x_ref, w_ref, idx_ref, gidx_ref):
    g = pl.program_id(0)
    xb = x_ref[0, 0, :, :]
    bidx = _argmin_codes(xb, w_ref)
    idx_ref[0, 0, :, :] = bidx
    gidx_ref[0, 0, :, :] = bidx + g * K


def _tc2_body(x_ref, zq_ref, w_ref, idx_ref, gidx_ref, q1_ref, s_ref):
    g = pl.program_id(0)
    b = pl.program_id(1)
    xb = x_ref[0, 0, :, :]
    zq = jnp.transpose(zq_ref[0, 0, :, :])
    diff = zq - xb
    q1 = xb + diff
    r = xb - q1

    @pl.when((g == 0) & (b == 0))
    def _init():
        s_ref[0, 0] = 0.0

    s_ref[0, 0] += jnp.sum(diff * diff)
    q1_ref[0, 0, :, :] = q1
    bidx = _argmin_codes(r, w_ref)
    idx_ref[0, 0, :, :] = bidx
    gidx_ref[0, 0, :, :] = bidx + g * K


def _tc3_body(x_ref, q1_ref, zq2_ref, qs_ref, s_ref):
    g = pl.program_id(0)
    b = pl.program_id(1)
    xb = x_ref[0, 0, :, :]
    q1 = q1_ref[0, 0, :, :]
    zq2 = jnp.transpose(zq2_ref[0, 0, :, :])
    r = xb - q1
    diff2 = zq2 - r
    q2 = r + diff2

    @pl.when((g == 0) & (b == 0))
    def _init():
        s_ref[0, 0] = 0.0

    s_ref[0, 0] += jnp.sum(diff2 * diff2)
    qs_ref[0, 0, :, :] = q1 + q2


_X_SPEC = pl.BlockSpec((1, 1, E, T), lambda g, b: (b, g, 0, 0))
_W_SPEC = pl.BlockSpec((1, K, E), lambda g, b: (g, 0, 0))
_ZQ_SPEC = pl.BlockSpec((1, 1, T, E), lambda g, b: (g, b, 0, 0))
_IDX_SPEC = pl.BlockSpec((1, 1, T, 1), lambda g, b: (g, b, 0, 0))
_Q_SPEC = pl.BlockSpec((1, 1, E, T), lambda g, b: (b, g, 0, 0))
_S_SPEC = pl.BlockSpec(memory_space=pltpu.SMEM, block_shape=(1, 1),
                       index_map=lambda g, b: (0, 0))

_IDX_TY = jax.ShapeDtypeStruct((G, BN, T, 1), jnp.int32)
_Q_TY = jax.ShapeDtypeStruct((BN, G, E, T), jnp.float32)
_S_TY = jax.ShapeDtypeStruct((1, 1), jnp.float32)


def _tc1(xg, w):
    return pl.pallas_call(
        _tc1_body,
        grid=(G, BN),
        in_specs=[_X_SPEC, _W_SPEC],
        out_specs=[_IDX_SPEC, _IDX_SPEC],
        out_shape=[_IDX_TY, _IDX_TY],
    )(xg, w)


def _tc2(xg, zq1, w2):
    return pl.pallas_call(
        _tc2_body,
        grid=(G, BN),
        in_specs=[_X_SPEC, _ZQ_SPEC, _W_SPEC],
        out_specs=[_IDX_SPEC, _IDX_SPEC, _Q_SPEC, _S_SPEC],
        out_shape=[_IDX_TY, _IDX_TY, _Q_TY, _S_TY],
    )(xg, zq1, w2)


def _tc3(xg, q1, zq2):
    return pl.pallas_call(
        _tc3_body,
        grid=(G, BN),
        in_specs=[_X_SPEC, _Q_SPEC, _ZQ_SPEC],
        out_specs=[_Q_SPEC, _S_SPEC],
        out_shape=[_Q_TY, _S_TY],
    )(xg, q1, zq2)


def _sc_gather_body(table_hbm, idx_hbm, out_hbm, idx_v, rows_v, sem):
    wid = lax.axis_index("s") * SC_CORES + lax.axis_index("c")
    base = wid * CH_PER_W
    pltpu.sync_copy(idx_hbm.at[pl.ds(base, CH_PER_W)], idx_v)
    copies = [
        pltpu.async_copy(table_hbm.at[idx_v.at[j]], rows_v.at[j], sem)
        for j in range(CH_PER_W)
    ]
    for cp in copies:
        cp.wait()
    pltpu.sync_copy(rows_v, out_hbm.at[pl.ds(base, CH_PER_W)])


def _sc_gather(table, gidx2d):
    call = pl.kernel(
        _sc_gather_body,
        out_type=jax.ShapeDtypeStruct((NW * CH_PER_W, IDX_CH, E), jnp.float32),
        mesh=plsc.VectorSubcoreMesh(core_axis_name="c", subcore_axis_name="s"),
        scratch_types=[
            pltpu.VMEM((CH_PER_W, IDX_CH), jnp.int32),
            pltpu.VMEM((CH_PER_W, IDX_CH, E), jnp.float32),
            pltpu.SemaphoreType.DMA,
        ],
        compiler_params=pltpu.CompilerParams(use_tc_tiling_on_sc=False),
    )
    return call(table, gidx2d)


def kernel(x, codebooks1, codebooks2):
    xg = x.reshape(BN, G, E, T)

    idx1, gidx1 = _tc1(xg, codebooks1)
    zq1 = _sc_gather(codebooks1.reshape(G * K, E),
                     gidx1.reshape(NW * CH_PER_W, IDX_CH))
    idx2, gidx2, q1, s1 = _tc2(xg, zq1.reshape(G, BN, T, E), codebooks2)
    zq2 = _sc_gather(codebooks2.reshape(G * K, E),
                     gidx2.reshape(NW * CH_PER_W, IDX_CH))
    qs, s2 = _tc3(xg, q1, zq2.reshape(G, BN, T, E))

    quantized_out = qs.reshape(BN, G * E, T)

    m1 = s1[0, 0] / NLOSS
    m2 = s2[0, 0] / NLOSS
    l1 = 1.0 * m1 + 0.25 * m1
    l2 = 1.0 * m2 + 0.25 * m2
    total_loss = (l1 + l2) / 2.0

    i1 = idx1.reshape(G, NTOK)
    i2 = idx2.reshape(G, NTOK)
    out = (quantized_out, total_loss)
    out += tuple(i1[g] for g in range(G))
    out += tuple(i2[g] for g in range(G))
    return out
